# Initial kernel scaffold; baseline (speedup 1.0000x reference)
import jax
import jax.numpy as jnp
from jax.experimental import pallas as pl


def kernel(x, item_table, brand_table, cat_table, price_w, price_b, fusion_w, fusion_b):
    raise NotImplementedError("write your pallas kernel here")


# speedup vs baseline: 5.5430x; 5.5430x over previous
import functools

import jax
import jax.numpy as jnp
from jax import lax
from jax.experimental import pallas as pl
from jax.experimental.pallas import tpu as pltpu
from jax.experimental.pallas import tpu_sc as plsc

_HI = jax.lax.Precision.HIGHEST


def _mm_body(t_ref, w_ref, o_ref):
    o_ref[...] = lax.dot(t_ref[...], w_ref[...], precision=_HI,
                         preferred_element_type=jnp.float32)


def _premul(table, w, blk_rows):
    rows, _ = table.shape
    d_out = w.shape[1]
    grid = rows // blk_rows
    return pl.pallas_call(
        _mm_body,
        grid=(grid,),
        in_specs=[
            pl.BlockSpec((blk_rows, table.shape[1]), lambda i: (i, 0)),
            pl.BlockSpec(w.shape, lambda i: (0, 0)),
        ],
        out_specs=pl.BlockSpec((blk_rows, d_out), lambda i: (i, 0)),
        out_shape=jax.ShapeDtypeStruct((rows, d_out), jnp.float32),
    )(table, w)


def _smalls_body(ct_ref, wc_ref, wp_ref, pw_ref, pb_ref, fb_ref, tc_ref, v_ref):
    cvec = jnp.sum(pb_ref[...] * wp_ref[...], axis=0, keepdims=True) + fb_ref[...]
    tc_ref[...] = lax.dot(ct_ref[...], wc_ref[...], precision=_HI,
                          preferred_element_type=jnp.float32) + cvec
    v_ref[...] = jnp.sum(pw_ref[...] * wp_ref[...], axis=0, keepdims=True)


def _sc_gather_add(it2d, br2d, ct1d, pr1d, ti, tb, tc, vrow, n, d):
    nc, ns = 2, 16
    nw = nc * ns
    rpw = n // nw
    k = 512
    g = rpw // k
    kb = k // 128
    nq = d // 16

    mesh = plsc.VectorSubcoreMesh(core_axis_name="c", subcore_axis_name="s")

    @functools.partial(
        pl.kernel,
        out_type=jax.ShapeDtypeStruct((n, d), jnp.float32),
        mesh=mesh,
        compiler_params=pltpu.CompilerParams(use_tc_tiling_on_sc=False),
        scratch_types=[
            pltpu.VMEM((k,), jnp.int32),
            pltpu.VMEM((k,), jnp.int32),
            pltpu.VMEM((k,), jnp.int32),
            pltpu.VMEM((k,), jnp.float32),
            pltpu.VMEM((k, d), jnp.float32),
            pltpu.VMEM((k, d), jnp.float32),
            pltpu.VMEM((tc.shape[0], d), jnp.float32),
            pltpu.VMEM((1, d), jnp.float32),
            pltpu.SemaphoreType.DMA,
        ],
    )
    def body(it_h, br_h, ct_h, pr_h, ti_h, tb_h, tc_h, v_h, out_h,
             itv, brv, ctv, prv, av, bv, tcv, vv, sem):
        wid = lax.axis_index("s") * nc + lax.axis_index("c")
        pltpu.sync_copy(tc_h, tcv)
        pltpu.sync_copy(v_h, vv)
        vqs = [vv[0, pl.ds(q * 16, 16)] for q in range(nq)]

        def chunk(gi, carry):
            base = wid * rpw + gi * k
            pltpu.sync_copy(it_h.at[pl.ds(base, k)], itv)
            pltpu.sync_copy(br_h.at[pl.ds(base, k)], brv)
            pltpu.sync_copy(ct_h.at[pl.ds(base, k)], ctv)
            pltpu.sync_copy(pr_h.at[pl.ds(base, k)], prv)
            cps = []
            for j in range(kb):
                cps.append(pltpu.async_copy(
                    ti_h.at[itv.at[pl.ds(j * 128, 128)]],
                    av.at[pl.ds(j * 128, 128)], sem))
                cps.append(pltpu.async_copy(
                    tb_h.at[brv.at[pl.ds(j * 128, 128)]],
                    bv.at[pl.ds(j * 128, 128)], sem))
            for cp in cps:
                cp.wait()

            def rowgroup(rg, c2):
                r0 = rg * 16
                ct16 = ctv[pl.ds(r0, 16)]
                p16 = prv[pl.ds(r0, 16)]
                for i in range(16):
                    ct_i = ct16[i]
                    pvec = jnp.full((16,), p16[i], jnp.float32)
                    r = r0 + i
                    for q in range(nq):
                        a = av[r, pl.ds(q * 16, 16)]
                        b = bv[r, pl.ds(q * 16, 16)]
                        c = tcv[ct_i, pl.ds(q * 16, 16)]
                        av[r, pl.ds(q * 16, 16)] = a + b + c + pvec * vqs[q]
                return c2

            lax.fori_loop(0, k // 16, rowgroup, 0)
            pltpu.sync_copy(av, out_h.at[pl.ds(base, k)])
            return carry

        lax.fori_loop(0, g, chunk, 0)

    return body(it2d, br2d, ct1d, pr1d, ti, tb, tc, vrow)


def kernel(x, item_table, brand_table, cat_table, price_w, price_b,
           fusion_w, fusion_b):
    n = x.shape[0]
    d_item = item_table.shape[1]
    d_brand = brand_table.shape[1]
    d_cat = cat_table.shape[1]
    d_out = fusion_w.shape[1]

    wi = fusion_w[:d_item]
    wb = fusion_w[d_item:d_item + d_brand]
    wc = fusion_w[d_item + d_brand:d_item + d_brand + d_cat]
    wp = fusion_w[d_item + d_brand + d_cat:]

    item_idx = x[:, 0].astype(jnp.int32)
    brand_idx = x[:, 1].astype(jnp.int32)
    cat_idx = x[:, 2].astype(jnp.int32)
    price = x[:, 3]

    ti = _premul(item_table, wi, 2000)
    tb = _premul(brand_table, wb, 2000)

    tc, vrow = pl.pallas_call(
        _smalls_body,
        in_specs=[pl.BlockSpec(a.shape, lambda: (0, 0)) for a in (
            cat_table, wc, wp, price_w.reshape(-1, 1),
            price_b.reshape(-1, 1), fusion_b.reshape(1, -1))],
        out_specs=[
            pl.BlockSpec((cat_table.shape[0], d_out), lambda: (0, 0)),
            pl.BlockSpec((1, d_out), lambda: (0, 0)),
        ],
        out_shape=[
            jax.ShapeDtypeStruct((cat_table.shape[0], d_out), jnp.float32),
            jax.ShapeDtypeStruct((1, d_out), jnp.float32),
        ],
    )(cat_table, wc, wp, price_w.reshape(-1, 1), price_b.reshape(-1, 1),
      fusion_b.reshape(1, -1))

    return _sc_gather_add(item_idx, brand_idx, cat_idx, price,
                          ti, tb, tc, vrow, n, d_out)


# speedup vs baseline: 6.6118x; 1.1928x over previous
import functools

import jax
import jax.numpy as jnp
from jax import lax
from jax.experimental import pallas as pl
from jax.experimental.pallas import tpu as pltpu
from jax.experimental.pallas import tpu_sc as plsc

_HI = jax.lax.Precision.HIGHEST


def _mm_body(t_ref, w_ref, o_ref):
    o_ref[...] = lax.dot(t_ref[...], w_ref[...], precision=_HI,
                         preferred_element_type=jnp.float32)


def _premul(table, w, blk_rows):
    rows, _ = table.shape
    d_out = w.shape[1]
    grid = rows // blk_rows
    return pl.pallas_call(
        _mm_body,
        grid=(grid,),
        in_specs=[
            pl.BlockSpec((blk_rows, table.shape[1]), lambda i: (i, 0)),
            pl.BlockSpec(w.shape, lambda i: (0, 0)),
        ],
        out_specs=pl.BlockSpec((blk_rows, d_out), lambda i: (i, 0)),
        out_shape=jax.ShapeDtypeStruct((rows, d_out), jnp.float32),
    )(table, w)


def _smalls_body(ct_ref, wc_ref, wp_ref, pw_ref, pb_ref, fb_ref, tc_ref, v_ref):
    cvec = jnp.sum(pb_ref[...] * wp_ref[...], axis=0, keepdims=True) + fb_ref[...]
    tc_ref[...] = lax.dot(ct_ref[...], wc_ref[...], precision=_HI,
                          preferred_element_type=jnp.float32) + cvec
    v_ref[...] = jnp.sum(pw_ref[...] * wp_ref[...], axis=0, keepdims=True)


def _sc_gather_add(it1d, br1d, ct1d, pr1d, ti, tb, tc, vrow, n, d):
    nc, ns = 2, 16
    nw = nc * ns
    rpw = n // nw
    k = 256
    g = rpw // k
    kb = k // 128
    nq = d // 16

    mesh = plsc.VectorSubcoreMesh(core_axis_name="c", subcore_axis_name="s")

    @functools.partial(
        pl.kernel,
        out_type=jax.ShapeDtypeStruct((n, d), jnp.float32),
        mesh=mesh,
        compiler_params=pltpu.CompilerParams(use_tc_tiling_on_sc=False),
        scratch_types=[
            pltpu.VMEM((2, k), jnp.int32),
            pltpu.VMEM((2, k), jnp.int32),
            pltpu.VMEM((2, k), jnp.int32),
            pltpu.VMEM((2, k), jnp.float32),
            pltpu.VMEM((2, k, d), jnp.float32),
            pltpu.VMEM((2, k, d), jnp.float32),
            pltpu.VMEM((2, k, d), jnp.float32),
            pltpu.VMEM((tc.shape[0], d), jnp.float32),
            pltpu.VMEM((1, d), jnp.float32),
            pltpu.SemaphoreType.DMA,
            pltpu.SemaphoreType.DMA,
            pltpu.SemaphoreType.DMA,
            pltpu.SemaphoreType.DMA,
            pltpu.SemaphoreType.DMA,
            pltpu.SemaphoreType.DMA,
        ],
    )
    def body(it_h, br_h, ct_h, pr_h, ti_h, tb_h, tc_h, v_h, out_h,
             itv, brv, ctv, prv, av, bv, ov, tcv, vv,
             si0, si1, sg0, sg1, sw0, sw1):
        si = (si0, si1)
        sg = (sg0, sg1)
        sw = (sw0, sw1)
        wid = lax.axis_index("s") * nc + lax.axis_index("c")
        base0 = wid * rpw
        pltpu.sync_copy(tc_h, tcv)
        pltpu.sync_copy(v_h, vv)
        vqs = [vv[0, pl.ds(q * 16, 16)] for q in range(nq)]

        def fire_idx(x, p):
            base = base0 + x * k
            pltpu.async_copy(it_h.at[pl.ds(base, k)], itv.at[p], si[p])
            pltpu.async_copy(br_h.at[pl.ds(base, k)], brv.at[p], si[p])
            pltpu.async_copy(ct_h.at[pl.ds(base, k)], ctv.at[p], si[p])
            pltpu.async_copy(pr_h.at[pl.ds(base, k)], prv.at[p], si[p])

        def wait_idx(p):
            pltpu.make_async_copy(it_h.at[pl.ds(0, k)], itv.at[p], si[p]).wait()
            pltpu.make_async_copy(br_h.at[pl.ds(0, k)], brv.at[p], si[p]).wait()
            pltpu.make_async_copy(ct_h.at[pl.ds(0, k)], ctv.at[p], si[p]).wait()
            pltpu.make_async_copy(pr_h.at[pl.ds(0, k)], prv.at[p], si[p]).wait()

        def fire_gathers(p):
            cps = []
            for j in range(kb):
                s = pl.ds(j * 128, 128)
                cps.append(pltpu.async_copy(
                    ti_h.at[itv.at[p].at[s]], av.at[p].at[s], sg[p]))
                cps.append(pltpu.async_copy(
                    tb_h.at[brv.at[p].at[s]], bv.at[p].at[s], sg[p]))
            return cps

        def fire_wb(x, p):
            base = base0 + x * k
            pltpu.async_copy(ov.at[p], out_h.at[pl.ds(base, k)], sw[p])

        def wait_wb(p):
            pltpu.make_async_copy(ov.at[p], out_h.at[pl.ds(0, k)], sw[p]).wait()

        def compute(p):
            def rowgroup(rg, c2):
                r0 = rg * 16
                ct16 = ctv[p, pl.ds(r0, 16)]
                p16 = prv[p, pl.ds(r0, 16)]
                for i in range(16):
                    ct_i = ct16[i]
                    pvec = jnp.full((16,), p16[i], jnp.float32)
                    r = r0 + i
                    for q in range(nq):
                        s = pl.ds(q * 16, 16)
                        ov[p, r, s] = (av[p, r, s] + bv[p, r, s]
                                       + tcv[ct_i, s] + pvec * vqs[q])
                return c2

            lax.fori_loop(0, k // 16, rowgroup, 0)

        def pair(a, first, last):
            wait_idx(0)
            ga = fire_gathers(0)
            wait_idx(1)
            gb = fire_gathers(1)
            for cp in ga:
                cp.wait()
            if not first:
                wait_wb(0)
            compute(0)
            fire_wb(a, 0)
            if not last:
                fire_idx(a + 2, 0)
            for cp in gb:
                cp.wait()
            if not first:
                wait_wb(1)
            compute(1)
            fire_wb(a + 1, 1)
            if not last:
                fire_idx(a + 3, 1)

        fire_idx(0, 0)
        fire_idx(1, 1)
        pair(0, True, False)

        def looped(m, c):
            pair(2 * m, False, False)
            return c

        lax.fori_loop(1, g // 2 - 1, looped, 0)
        pair(g - 2, False, True)
        wait_wb(0)
        wait_wb(1)

    return body(it1d, br1d, ct1d, pr1d, ti, tb, tc, vrow)


def kernel(x, item_table, brand_table, cat_table, price_w, price_b,
           fusion_w, fusion_b):
    n = x.shape[0]
    d_item = item_table.shape[1]
    d_brand = brand_table.shape[1]
    d_cat = cat_table.shape[1]
    d_out = fusion_w.shape[1]

    wi = fusion_w[:d_item]
    wb = fusion_w[d_item:d_item + d_brand]
    wc = fusion_w[d_item + d_brand:d_item + d_brand + d_cat]
    wp = fusion_w[d_item + d_brand + d_cat:]

    item_idx = x[:, 0].astype(jnp.int32)
    brand_idx = x[:, 1].astype(jnp.int32)
    cat_idx = x[:, 2].astype(jnp.int32)
    price = x[:, 3]

    ti = _premul(item_table, wi, 2000)
    tb = _premul(brand_table, wb, 2000)

    tc, vrow = pl.pallas_call(
        _smalls_body,
        in_specs=[pl.BlockSpec(a.shape, lambda: (0, 0)) for a in (
            cat_table, wc, wp, price_w.reshape(-1, 1),
            price_b.reshape(-1, 1), fusion_b.reshape(1, -1))],
        out_specs=[
            pl.BlockSpec((cat_table.shape[0], d_out), lambda: (0, 0)),
            pl.BlockSpec((1, d_out), lambda: (0, 0)),
        ],
        out_shape=[
            jax.ShapeDtypeStruct((cat_table.shape[0], d_out), jnp.float32),
            jax.ShapeDtypeStruct((1, d_out), jnp.float32),
        ],
    )(cat_table, wc, wp, price_w.reshape(-1, 1), price_b.reshape(-1, 1),
      fusion_b.reshape(1, -1))

    return _sc_gather_add(item_idx, brand_idx, cat_idx, price,
                          ti, tb, tc, vrow, n, d_out)


# speedup vs baseline: 6.7005x; 1.0134x over previous
import functools

import jax
import jax.numpy as jnp
from jax import lax
from jax.experimental import pallas as pl
from jax.experimental.pallas import tpu as pltpu
from jax.experimental.pallas import tpu_sc as plsc

_HI = jax.lax.Precision.HIGHEST


def _mm_body(t_ref, w_ref, o_ref):
    o_ref[...] = lax.dot(t_ref[...], w_ref[...], precision=_HI,
                         preferred_element_type=jnp.float32)


def _premul(table, w, blk_rows):
    rows, _ = table.shape
    d_out = w.shape[1]
    grid = rows // blk_rows
    return pl.pallas_call(
        _mm_body,
        grid=(grid,),
        in_specs=[
            pl.BlockSpec((blk_rows, table.shape[1]), lambda i: (i, 0)),
            pl.BlockSpec(w.shape, lambda i: (0, 0)),
        ],
        out_specs=pl.BlockSpec((blk_rows, d_out), lambda i: (i, 0)),
        out_shape=jax.ShapeDtypeStruct((rows, d_out), jnp.float32),
    )(table, w)


def _smalls_body(ct_ref, wc_ref, wp_ref, pw_ref, pb_ref, fb_ref, tc_ref, v_ref):
    cvec = jnp.sum(pb_ref[...] * wp_ref[...], axis=0, keepdims=True) + fb_ref[...]
    tc_ref[...] = lax.dot(ct_ref[...], wc_ref[...], precision=_HI,
                          preferred_element_type=jnp.float32) + cvec
    v_ref[...] = jnp.sum(pw_ref[...] * wp_ref[...], axis=0, keepdims=True)


def _sc_gather_add(xt, ti, tb, tc, vrow, n, d):
    nc, ns = 2, 16
    nw = nc * ns
    rpw = n // nw
    k = 256
    g = rpw // k
    kb = k // 128
    nq = d // 16

    mesh = plsc.VectorSubcoreMesh(core_axis_name="c", subcore_axis_name="s")

    @functools.partial(
        pl.kernel,
        out_type=jax.ShapeDtypeStruct((n, d), jnp.float32),
        mesh=mesh,
        compiler_params=pltpu.CompilerParams(use_tc_tiling_on_sc=False),
        scratch_types=[
            pltpu.VMEM((2, k), jnp.float32),
            pltpu.VMEM((2, k), jnp.float32),
            pltpu.VMEM((2, k), jnp.float32),
            pltpu.VMEM((2, k), jnp.float32),
            pltpu.VMEM((2, k), jnp.int32),
            pltpu.VMEM((2, k), jnp.int32),
            pltpu.VMEM((2, k, d), jnp.float32),
            pltpu.VMEM((2, k, d), jnp.float32),
            pltpu.VMEM((2, k, d), jnp.float32),
            pltpu.VMEM((tc.shape[0], d), jnp.float32),
            pltpu.VMEM((1, d), jnp.float32),
            pltpu.SemaphoreType.DMA,
            pltpu.SemaphoreType.DMA,
            pltpu.SemaphoreType.DMA,
            pltpu.SemaphoreType.DMA,
            pltpu.SemaphoreType.DMA,
            pltpu.SemaphoreType.DMA,
        ],
    )
    def body(x_h, ti_h, tb_h, tc_h, v_h, out_h,
             xvi, xvb, xvc, xvp, itv, brv, av, bv, ov, tcv, vv,
             si0, si1, sg0, sg1, sw0, sw1):
        si = (si0, si1)
        sg = (sg0, sg1)
        sw = (sw0, sw1)
        wid = lax.axis_index("s") * nc + lax.axis_index("c")
        base0 = wid * rpw
        pltpu.sync_copy(tc_h, tcv)
        pltpu.sync_copy(v_h, vv)
        vqs = [vv[0, pl.ds(q * 16, 16)] for q in range(nq)]
        def fire_idx(x, p):
            base = base0 + x * k
            pltpu.async_copy(x_h.at[0, pl.ds(base, k)], xvi.at[p], si[p])
            pltpu.async_copy(x_h.at[1, pl.ds(base, k)], xvb.at[p], si[p])
            pltpu.async_copy(x_h.at[2, pl.ds(base, k)], xvc.at[p], si[p])
            pltpu.async_copy(x_h.at[3, pl.ds(base, k)], xvp.at[p], si[p])

        def wait_idx(p):
            pltpu.make_async_copy(x_h.at[0, pl.ds(0, k)], xvi.at[p], si[p]).wait()
            pltpu.make_async_copy(x_h.at[1, pl.ds(0, k)], xvb.at[p], si[p]).wait()
            pltpu.make_async_copy(x_h.at[2, pl.ds(0, k)], xvc.at[p], si[p]).wait()
            pltpu.make_async_copy(x_h.at[3, pl.ds(0, k)], xvp.at[p], si[p]).wait()

        def convert_idx(p):
            def grp(rg, cc):
                s = pl.ds(rg * 16, 16)
                itv[p, s] = xvi[p, s].astype(jnp.int32)
                brv[p, s] = xvb[p, s].astype(jnp.int32)
                return cc
            lax.fori_loop(0, k // 16, grp, 0)

        def fire_gathers(p):
            cps = []
            for j in range(kb):
                s = pl.ds(j * 128, 128)
                cps.append(pltpu.async_copy(
                    ti_h.at[itv.at[p].at[s]], av.at[p].at[s], sg[p]))
                cps.append(pltpu.async_copy(
                    tb_h.at[brv.at[p].at[s]], bv.at[p].at[s], sg[p]))
            return cps

        def fire_wb(x, p):
            base = base0 + x * k
            pltpu.async_copy(ov.at[p], out_h.at[pl.ds(base, k)], sw[p])

        def wait_wb(p):
            pltpu.make_async_copy(ov.at[p], out_h.at[pl.ds(0, k)], sw[p]).wait()

        def compute(p):
            def rowgroup(rg, cc):
                r0 = rg * 16
                ct16 = xvc[p, pl.ds(r0, 16)].astype(jnp.int32)
                p16 = xvp[p, pl.ds(r0, 16)]
                for i in range(16):
                    ct_i = ct16[i]
                    pvec = jnp.full((16,), p16[i], jnp.float32)
                    r = r0 + i
                    for q in range(nq):
                        s = pl.ds(q * 16, 16)
                        ov[p, r, s] = (av[p, r, s] + bv[p, r, s]
                                       + tcv[ct_i, s] + pvec * vqs[q])
                return cc

            lax.fori_loop(0, k // 16, rowgroup, 0)

        def pair(a, first, last):
            wait_idx(0)
            convert_idx(0)
            ga = fire_gathers(0)
            wait_idx(1)
            convert_idx(1)
            gb = fire_gathers(1)
            for cp in ga:
                cp.wait()
            if not first:
                wait_wb(0)
            compute(0)
            fire_wb(a, 0)
            if not last:
                fire_idx(a + 2, 0)
            for cp in gb:
                cp.wait()
            if not first:
                wait_wb(1)
            compute(1)
            fire_wb(a + 1, 1)
            if not last:
                fire_idx(a + 3, 1)

        fire_idx(0, 0)
        fire_idx(1, 1)
        pair(0, True, False)

        def looped(m, c):
            pair(2 * m, False, False)
            return c

        lax.fori_loop(1, g // 2 - 1, looped, 0)
        pair(g - 2, False, True)
        wait_wb(0)
        wait_wb(1)

    return body(xt, ti, tb, tc, vrow)


def kernel(x, item_table, brand_table, cat_table, price_w, price_b,
           fusion_w, fusion_b):
    n = x.shape[0]
    d_item = item_table.shape[1]
    d_brand = brand_table.shape[1]
    d_cat = cat_table.shape[1]
    d_out = fusion_w.shape[1]

    wi = fusion_w[:d_item]
    wb = fusion_w[d_item:d_item + d_brand]
    wc = fusion_w[d_item + d_brand:d_item + d_brand + d_cat]
    wp = fusion_w[d_item + d_brand + d_cat:]

    ti = _premul(item_table, wi, 2000)
    tb = _premul(brand_table, wb, 2000)

    tc, vrow = pl.pallas_call(
        _smalls_body,
        in_specs=[pl.BlockSpec(a.shape, lambda: (0, 0)) for a in (
            cat_table, wc, wp, price_w.reshape(-1, 1),
            price_b.reshape(-1, 1), fusion_b.reshape(1, -1))],
        out_specs=[
            pl.BlockSpec((cat_table.shape[0], d_out), lambda: (0, 0)),
            pl.BlockSpec((1, d_out), lambda: (0, 0)),
        ],
        out_shape=[
            jax.ShapeDtypeStruct((cat_table.shape[0], d_out), jnp.float32),
            jax.ShapeDtypeStruct((1, d_out), jnp.float32),
        ],
    )(cat_table, wc, wp, price_w.reshape(-1, 1), price_b.reshape(-1, 1),
      fusion_b.reshape(1, -1))

    return _sc_gather_add(x.T, ti, tb, tc, vrow, n, d_out)
